# baseline (speedup 1.0000x reference)
import functools

import jax
import jax.numpy as jnp
from jax import lax
from jax.experimental import pallas as pl
from jax.experimental.pallas import tpu as pltpu
from jax.experimental.pallas import tpu_sc as plsc

H = 128
NGRAPH = 32


def _silu(x):
    return x * jax.nn.sigmoid(x)


def _pad_rows(a, m, fill=0):
    n = a.shape[0]
    r = (-n) % m
    if r == 0:
        return a
    pad = [(0, r)] + [(0, 0)] * (a.ndim - 1)
    return jnp.pad(a, pad, constant_values=fill)


def _edge_mlp_body(gs, gd, inv, w1a, w1b, w1c, b1, w2, b2, wiv, bi, out):
    x = (jnp.dot(gs[...], w1a[...], preferred_element_type=jnp.float32)
         + jnp.dot(gd[...], w1b[...], preferred_element_type=jnp.float32)
         + jnp.dot(inv[...], w1c[...], preferred_element_type=jnp.float32)
         + b1[...])
    m = _silu(x)
    m = _silu(jnp.dot(m, w2[...], preferred_element_type=jnp.float32) + b2[...])
    g = jax.nn.sigmoid(jnp.sum(m * wiv[...], axis=1, keepdims=True) + bi[0, 0])
    out[...] = m * g


def _edge_mlp(gs, gd, inv8, mlp, inf, block=1024):
    e = gs.shape[0]
    w1, b1, w2, b2 = mlp
    wi, bi = inf
    grid = (e // block,)
    full = lambda shape: pl.BlockSpec(shape, lambda i: (0, 0))
    return pl.pallas_call(
        _edge_mlp_body,
        grid=grid,
        in_specs=[
            pl.BlockSpec((block, H), lambda i: (i, 0)),
            pl.BlockSpec((block, H), lambda i: (i, 0)),
            pl.BlockSpec((block, 8), lambda i: (i, 0)),
            full((H, H)), full((H, H)), full((8, H)), full((1, H)),
            full((H, H)), full((1, H)),
            full((1, H)), full((1, 1)),
        ],
        out_specs=pl.BlockSpec((block, H), lambda i: (i, 0)),
        out_shape=jax.ShapeDtypeStruct((e, H), jnp.float32),
    )(gs, gd, inv8,
      w1[:H], w1[H:2 * H], _pad_rows(w1[2 * H:], 8), b1[None, :],
      w2, b2[None, :], wi.T, bi[None, :])


def _embed_body(x, we, be, out):
    out[...] = jnp.dot(x[...], we[...], preferred_element_type=jnp.float32) + be[...]


def _embed(x, we, be, block=2048):
    n = x.shape[0]
    xp = _pad_rows(x, block)
    grid = (xp.shape[0] // block,)
    out = pl.pallas_call(
        _embed_body,
        grid=grid,
        in_specs=[
            pl.BlockSpec((block, H), lambda i: (i, 0)),
            pl.BlockSpec((H, H), lambda i: (0, 0)),
            pl.BlockSpec((1, H), lambda i: (0, 0)),
        ],
        out_specs=pl.BlockSpec((block, H), lambda i: (i, 0)),
        out_shape=jax.ShapeDtypeStruct((xp.shape[0], H), jnp.float32),
    )(xp, we, be[None, :])
    return out[:n]


def _update_body(h, agg_a, agg_b, w1a, w1b, b1, w2, b2, out):
    agg = agg_a[...] + agg_b[...]
    x = (jnp.dot(h[...], w1a[...], preferred_element_type=jnp.float32)
         + jnp.dot(agg, w1b[...], preferred_element_type=jnp.float32)
         + b1[...])
    out[...] = h[...] + jnp.dot(_silu(x), w2[...], preferred_element_type=jnp.float32) + b2[...]


def _update(h, agg2, upd, block=2048):
    n = h.shape[0]
    nrow = agg2.shape[0] // 2
    off = nrow // block
    w1, b1, w2, b2 = upd
    hp = _pad_rows(h, block)
    grid = (hp.shape[0] // block,)
    full = lambda shape: pl.BlockSpec(shape, lambda i: (0, 0))
    out = pl.pallas_call(
        _update_body,
        grid=grid,
        in_specs=[
            pl.BlockSpec((block, H), lambda i: (i, 0)),
            pl.BlockSpec((block, H), lambda i: (i, 0)),
            pl.BlockSpec((block, H), lambda i: (i + off, 0)),
            full((H, H)), full((H, H)), full((1, H)), full((H, H)), full((1, H)),
        ],
        out_specs=pl.BlockSpec((block, H), lambda i: (i, 0)),
        out_shape=jax.ShapeDtypeStruct((hp.shape[0], H), jnp.float32),
    )(hp, agg2, agg2, w1[:H], w1[H:], b1[None, :], w2, b2[None, :])
    return out[:n]


def _prepool_body(h, bid, w1, b1, w2, b2, out):
    i = pl.program_id(0)

    @pl.when(i == 0)
    def _():
        out[...] = jnp.zeros_like(out)

    x = _silu(jnp.dot(h[...], w1[...], preferred_element_type=jnp.float32) + b1[...])
    x = jnp.dot(x, w2[...], preferred_element_type=jnp.float32) + b2[...]
    ids = bid[...]
    lanes = jnp.arange(NGRAPH, dtype=jnp.int32)[None, :].astype(jnp.float32)
    onehot = (ids == lanes).astype(jnp.float32)
    out[...] += jax.lax.dot_general(onehot, x, (((0,), (0,)), ((), ())),
                                    preferred_element_type=jnp.float32)


def _prepool(h, bid, pre, block=2048):
    w1, b1, w2, b2 = pre
    hp = _pad_rows(h, block)
    bidp = _pad_rows(bid.astype(jnp.float32)[:, None], block, fill=NGRAPH + 1)
    grid = (hp.shape[0] // block,)
    full = lambda shape: pl.BlockSpec(shape, lambda i: (0, 0))
    return pl.pallas_call(
        _prepool_body,
        grid=grid,
        in_specs=[
            pl.BlockSpec((block, H), lambda i: (i, 0)),
            pl.BlockSpec((block, 1), lambda i: (i, 0)),
            full((H, H)), full((1, H)), full((H, H)), full((1, H)),
        ],
        out_specs=pl.BlockSpec((NGRAPH, H), lambda i: (0, 0)),
        out_shape=jax.ShapeDtypeStruct((NGRAPH, H), jnp.float32),
    )(hp, bidp, w1, b1[None, :], w2, b2[None, :])


def _postpool_body(st, w1, b1, w2v, b2, out):
    x = _silu(jnp.dot(st[...], w1[...], preferred_element_type=jnp.float32) + b1[...])
    r = jnp.sum(x * w2v[...], axis=1, keepdims=True) + b2[0, 0]
    out[...] = jnp.broadcast_to(r, (NGRAPH, H))


def _postpool(state, post):
    w1, b1, w2, b2 = post
    out = pl.pallas_call(
        _postpool_body,
        in_specs=[
            pl.BlockSpec((NGRAPH, 3 * H), lambda: (0, 0)),
            pl.BlockSpec((3 * H, H), lambda: (0, 0)),
            pl.BlockSpec((1, H), lambda: (0, 0)),
            pl.BlockSpec((1, H), lambda: (0, 0)),
            pl.BlockSpec((1, 1), lambda: (0, 0)),
        ],
        out_specs=pl.BlockSpec((NGRAPH, H), lambda: (0, 0)),
        out_shape=jax.ShapeDtypeStruct((NGRAPH, H), jnp.float32),
    )(state, w1, b1[None, :], w2.T, b2[None, :])
    return out[:, 0]


_GK = 128


def _sc_gather2(table_a, idx_a, table_b, idx_b):
    e = idx_a.shape[0]
    assert e % _GK == 0 and e == idx_b.shape[0]
    steps = e // _GK
    nw = 32
    per_w = -(-steps // nw)
    mesh = plsc.VectorSubcoreMesh(core_axis_name="c", subcore_axis_name="s")

    @functools.partial(
        pl.kernel, mesh=mesh,
        out_type=(jax.ShapeDtypeStruct((e, H), jnp.float32),
                  jax.ShapeDtypeStruct((e, H), jnp.float32)),
        scratch_types=[
            pltpu.VMEM((_GK,), jnp.int32), pltpu.VMEM((_GK, H), jnp.float32),
            pltpu.VMEM((_GK,), jnp.int32), pltpu.VMEM((_GK, H), jnp.float32),
            pltpu.SemaphoreType.DMA, pltpu.SemaphoreType.DMA,
        ],
    )
    def gk(ta, ia, tb, ib, oa, ob, iva, rva, ivb, rvb, sema, semb):
        wid = lax.axis_index("s") * 2 + lax.axis_index("c")

        def body(j, carry):
            s = wid + j * nw

            @pl.when(s < steps)
            def _():
                base = s * _GK
                pltpu.sync_copy(ia.at[pl.ds(base, _GK)], iva)
                pltpu.sync_copy(ib.at[pl.ds(base, _GK)], ivb)
                ca = pltpu.async_copy(ta.at[iva], rva, sema)
                cb = pltpu.async_copy(tb.at[ivb], rvb, semb)
                ca.wait()
                pltpu.sync_copy(rva, oa.at[pl.ds(base, _GK)])
                cb.wait()
                pltpu.sync_copy(rvb, ob.at[pl.ds(base, _GK)])

            return carry

        lax.fori_loop(0, per_w, body, 0)

    return gk(table_a, idx_a, table_b, idx_b)


def _segsum(m, dst, n):
    return jax.ops.segment_sum(m, dst, num_segments=n)


_CH = 8192
_ACC = _CH + 256


def _read_off(of_ref, i):
    v = plsc.load_gather(of_ref, [jnp.full((16,), i, jnp.int32)])
    return jnp.max(v)


def _sc_segsum2(nch, ma, dla, ofa, mb, dlb, ofb):
    mesh = plsc.VectorSubcoreMesh(core_axis_name="c", subcore_axis_name="s")
    nrow = nch * _CH
    zrows = jnp.zeros((_ACC // 16, H), jnp.float32)

    @functools.partial(
        pl.kernel, mesh=mesh,
        out_type=jax.ShapeDtypeStruct((2 * nrow, H), jnp.float32),
        scratch_types=[
            pltpu.VMEM((_GK,), jnp.int32), pltpu.VMEM((_GK, H), jnp.float32),
            pltpu.VMEM((32,), jnp.int32), pltpu.VMEM((32,), jnp.int32),
            pltpu.VMEM_SHARED((_ACC, H), jnp.float32),
        ],
        compiler_params=pltpu.CompilerParams(needs_layout_passes=False),
    )
    def sk(ma_r, dla_r, ofa_r, mb_r, dlb_r, ofb_r, zr, oo,
           idxv, payv, ofav, ofbv, acc):
        core = lax.axis_index("c")
        sub = lax.axis_index("s")
        wid = sub * 2 + core
        pltpu.sync_copy(ofa_r, ofav)
        pltpu.sync_copy(ofb_r, ofbv)

        def chunk(c, carry):
            b = sub * (_ACC // 16)
            pltpu.sync_copy(zr, acc.at[pl.ds(b, _ACC // 16)])
            plsc.subcore_barrier()

            for m_r, dl_r, of_r in ((ma_r, dla_r, ofav), (mb_r, dlb_r, ofbv)):
                s1 = _read_off(of_r, c + 1)

                def sbody(u):
                    base = u * _GK
                    pltpu.sync_copy(dl_r.at[pl.ds(base, _GK)], idxv)
                    pltpu.sync_copy(m_r.at[pl.ds(base, _GK)], payv)
                    pltpu.sync_copy(payv, acc.at[idxv], add=True)
                    return u + 32

                lax.while_loop(lambda u: u < s1, sbody, _read_off(of_r, c) + wid)
            plsc.subcore_barrier()

            rb = sub * (_CH // 16)
            pltpu.sync_copy(acc.at[pl.ds(rb, _CH // 16)],
                            oo.at[pl.ds(core * nrow + c * _CH + rb, _CH // 16)])
            plsc.subcore_barrier()
            return carry

        lax.fori_loop(0, nch, chunk, 0)

    return sk(ma, dla, ofa, mb, dlb, ofb, zrows)


def _bucket_permute(src, dst, inv, nch, e_pad):
    e = dst.shape[0]
    bucket = jax.lax.shift_right_logical(dst, 13)
    oh = (bucket[:, None] == jnp.arange(nch, dtype=jnp.int32)[None, :]).astype(jnp.int32)
    cum = jnp.cumsum(oh, axis=0)
    within = jnp.take_along_axis(cum, bucket[:, None], axis=1)[:, 0] - 1
    counts = cum[-1]
    pcounts = ((counts + _GK - 1) // _GK) * _GK
    offs = jnp.concatenate([jnp.zeros((1,), jnp.int32),
                            jnp.cumsum(pcounts).astype(jnp.int32)])
    pos = offs[bucket] + within
    perm = jnp.zeros((e_pad,), jnp.int32).at[pos].set(jnp.arange(e, dtype=jnp.int32))
    valid = jnp.zeros((e_pad,), jnp.bool_).at[pos].set(True)
    sp = src[perm]
    dp = dst[perm]
    srcp = jnp.where(valid, sp, 0)
    dstg = jnp.where(valid, dp, 0)
    dloc = jnp.where(valid, dp & (_CH - 1), _CH)
    invp = jnp.where(valid[:, None], inv[perm], 0.0)
    step_offs = offs // _GK
    step_offs = jnp.pad(step_offs, (0, 32 - nch - 1), mode='edge')
    return srcp, dstg, dloc, invp, step_offs


def kernel(features_rank_0, features_rank_1, features_rank_2,
           adjacencies_rank_0, adjacencies_rank_1,
           incidences_rank_0, incidences_rank_1,
           inv_rr_rank_0, inv_rr_rank_1,
           inv_rrm1_rank_0, inv_rrm1_rank_1,
           batch_rank_0, batch_rank_1, batch_rank_2, params):
    n0 = features_rank_0.shape[0]
    n1 = features_rank_1.shape[0]
    n2 = features_rank_2.shape[0]
    sizes = {'rank_0': n0, 'rank_1': n1, 'rank_2': n2}

    we, be = params['embed']
    h = {
        'rank_0': _embed(features_rank_0, we, be),
        'rank_1': _embed(features_rank_1, we, be),
        'rank_2': _embed(features_rank_2, we, be),
    }

    eb = 1024
    adj = {'rank_0': adjacencies_rank_0, 'rank_1': adjacencies_rank_1}
    inc = {'rank_0': incidences_rank_0, 'rank_1': incidences_rank_1}
    inv_rr = {'rank_0': inv_rr_rank_0, 'rank_1': inv_rr_rank_1}
    inv_rm = {'rank_0': inv_rrm1_rank_0, 'rank_1': inv_rrm1_rank_1}
    upper = {'rank_0': 'rank_1', 'rank_1': 'rank_2'}
    nch = {'rank_0': 2, 'rank_1': 20}

    def prep(edges, inv, nc):
        e = edges.shape[1]
        e_pad = -(-(e + nc * _GK) // eb) * eb
        srcp, dstg, dloc, invp, offs = _bucket_permute(
            edges[0], edges[1], inv, nc, e_pad)
        inv8 = jnp.pad(invp, ((0, 0), (0, 8 - invp.shape[1])))
        return srcp, dstg, dloc, inv8, offs

    prepped = {}
    for r in ('rank_0', 'rank_1'):
        prepped[(r, 'adj')] = prep(adj[r], inv_rr[r], nch[r])
        prepped[(r, 'inc')] = prep(inc[r], inv_rm[r], nch[r])

    for lp in params['layers']:
        h_new = dict(h)
        for r in ('rank_0', 'rank_1'):
            p = lp[r]
            src, dstg, dloc0, inv8, offs0 = prepped[(r, 'adj')]
            gs, gd = _sc_gather2(h[r], src, h[r], dstg)
            m0 = _edge_mlp(gs, gd, inv8, p['msg_adj'], p['inf_adj'], block=eb)
            src2, dstg2, dloc1, inv8b, offs1 = prepped[(r, 'inc')]
            gs2, gd2 = _sc_gather2(h[upper[r]], src2, h[r], dstg2)
            m1 = _edge_mlp(gs2, gd2, inv8b, p['msg_inc'], p['inf_inc'], block=eb)
            agg2 = _sc_segsum2(nch[r], m0, dloc0, offs0, m1, dloc1, offs1)
            h_new[r] = _update(h[r], agg2, p['upd'])
        h = h_new

    batches = {'rank_0': batch_rank_0, 'rank_1': batch_rank_1, 'rank_2': batch_rank_2}
    pooled = [_prepool(h[r], batches[r], params['pre_pool'][r])
              for r in ('rank_0', 'rank_1', 'rank_2')]
    state = jnp.concatenate(pooled, axis=1)
    return _postpool(state, params['post_pool'])


# speedup vs baseline: 1.7911x; 1.7911x over previous
import functools

import jax
import jax.numpy as jnp
from jax import lax
from jax.experimental import pallas as pl
from jax.experimental.pallas import tpu as pltpu
from jax.experimental.pallas import tpu_sc as plsc

H = 128
NGRAPH = 32


def _silu(x):
    return x * jax.nn.sigmoid(x)


def _pad_rows(a, m, fill=0):
    n = a.shape[0]
    r = (-n) % m
    if r == 0:
        return a
    pad = [(0, r)] + [(0, 0)] * (a.ndim - 1)
    return jnp.pad(a, pad, constant_values=fill)


def _edge_mlp_body(gs, gd, inv, w1a, w1b, w1c, b1, w2, b2, wiv, bi, out):
    x = (jnp.dot(gs[...], w1a[...], preferred_element_type=jnp.float32)
         + jnp.dot(gd[...], w1b[...], preferred_element_type=jnp.float32)
         + jnp.dot(inv[...], w1c[...], preferred_element_type=jnp.float32)
         + b1[...])
    m = _silu(x)
    m = _silu(jnp.dot(m, w2[...], preferred_element_type=jnp.float32) + b2[...])
    g = jax.nn.sigmoid(jnp.sum(m * wiv[...], axis=1, keepdims=True) + bi[0, 0])
    out[...] = m * g


def _edge_mlp(gs, gd, inv8, mlp, inf, block=1024):
    e = gs.shape[0]
    w1, b1, w2, b2 = mlp
    wi, bi = inf
    grid = (e // block,)
    full = lambda shape: pl.BlockSpec(shape, lambda i: (0, 0))
    return pl.pallas_call(
        _edge_mlp_body,
        grid=grid,
        in_specs=[
            pl.BlockSpec((block, H), lambda i: (i, 0)),
            pl.BlockSpec((block, H), lambda i: (i, 0)),
            pl.BlockSpec((block, 8), lambda i: (i, 0)),
            full((H, H)), full((H, H)), full((8, H)), full((1, H)),
            full((H, H)), full((1, H)),
            full((1, H)), full((1, 1)),
        ],
        out_specs=pl.BlockSpec((block, H), lambda i: (i, 0)),
        out_shape=jax.ShapeDtypeStruct((e, H), jnp.float32),
    )(gs, gd, inv8,
      w1[:H], w1[H:2 * H], _pad_rows(w1[2 * H:], 8), b1[None, :],
      w2, b2[None, :], wi.T, bi[None, :])


def _embed_body(x, we, be, out):
    out[...] = jnp.dot(x[...], we[...], preferred_element_type=jnp.float32) + be[...]


def _embed(x, we, be, block=2048):
    n = x.shape[0]
    xp = _pad_rows(x, block)
    grid = (xp.shape[0] // block,)
    out = pl.pallas_call(
        _embed_body,
        grid=grid,
        in_specs=[
            pl.BlockSpec((block, H), lambda i: (i, 0)),
            pl.BlockSpec((H, H), lambda i: (0, 0)),
            pl.BlockSpec((1, H), lambda i: (0, 0)),
        ],
        out_specs=pl.BlockSpec((block, H), lambda i: (i, 0)),
        out_shape=jax.ShapeDtypeStruct((xp.shape[0], H), jnp.float32),
    )(xp, we, be[None, :])
    return out[:n]


def _update_body(h, agg, w1a, w1b, b1, w2, b2, out):
    x = (jnp.dot(h[...], w1a[...], preferred_element_type=jnp.float32)
         + jnp.dot(agg[...], w1b[...], preferred_element_type=jnp.float32)
         + b1[...])
    out[...] = h[...] + jnp.dot(_silu(x), w2[...], preferred_element_type=jnp.float32) + b2[...]


def _update_single(h, agg, upd, block=2048):
    n = h.shape[0]
    w1, b1, w2, b2 = upd
    hp = _pad_rows(h, block)
    aggp = _pad_rows(agg, block)
    grid = (hp.shape[0] // block,)
    full = lambda shape: pl.BlockSpec(shape, lambda i: (0, 0))
    out = pl.pallas_call(
        _update_body,
        grid=grid,
        in_specs=[
            pl.BlockSpec((block, H), lambda i: (i, 0)),
            pl.BlockSpec((block, H), lambda i: (i, 0)),
            full((H, H)), full((H, H)), full((1, H)), full((H, H)), full((1, H)),
        ],
        out_specs=pl.BlockSpec((block, H), lambda i: (i, 0)),
        out_shape=jax.ShapeDtypeStruct((hp.shape[0], H), jnp.float32),
    )(hp, aggp, w1[:H], w1[H:], b1[None, :], w2, b2[None, :])
    return out[:n]


def _prepool_body(h, bid, w1, b1, w2, b2, out):
    i = pl.program_id(0)

    @pl.when(i == 0)
    def _():
        out[...] = jnp.zeros_like(out)

    x = _silu(jnp.dot(h[...], w1[...], preferred_element_type=jnp.float32) + b1[...])
    x = jnp.dot(x, w2[...], preferred_element_type=jnp.float32) + b2[...]
    ids = bid[...]
    lanes = jnp.arange(NGRAPH, dtype=jnp.int32)[None, :].astype(jnp.float32)
    onehot = (ids == lanes).astype(jnp.float32)
    out[...] += jax.lax.dot_general(onehot, x, (((0,), (0,)), ((), ())),
                                    preferred_element_type=jnp.float32)


def _prepool(h, bid, pre, block=2048):
    w1, b1, w2, b2 = pre
    hp = _pad_rows(h, block)
    bidp = _pad_rows(bid.astype(jnp.float32)[:, None], block, fill=NGRAPH + 1)
    grid = (hp.shape[0] // block,)
    full = lambda shape: pl.BlockSpec(shape, lambda i: (0, 0))
    return pl.pallas_call(
        _prepool_body,
        grid=grid,
        in_specs=[
            pl.BlockSpec((block, H), lambda i: (i, 0)),
            pl.BlockSpec((block, 1), lambda i: (i, 0)),
            full((H, H)), full((1, H)), full((H, H)), full((1, H)),
        ],
        out_specs=pl.BlockSpec((NGRAPH, H), lambda i: (0, 0)),
        out_shape=jax.ShapeDtypeStruct((NGRAPH, H), jnp.float32),
    )(hp, bidp, w1, b1[None, :], w2, b2[None, :])


def _postpool_body(st, w1, b1, w2v, b2, out):
    x = _silu(jnp.dot(st[...], w1[...], preferred_element_type=jnp.float32) + b1[...])
    r = jnp.sum(x * w2v[...], axis=1, keepdims=True) + b2[0, 0]
    out[...] = jnp.broadcast_to(r, (NGRAPH, H))


def _postpool(state, post):
    w1, b1, w2, b2 = post
    out = pl.pallas_call(
        _postpool_body,
        in_specs=[
            pl.BlockSpec((NGRAPH, 3 * H), lambda: (0, 0)),
            pl.BlockSpec((3 * H, H), lambda: (0, 0)),
            pl.BlockSpec((1, H), lambda: (0, 0)),
            pl.BlockSpec((1, H), lambda: (0, 0)),
            pl.BlockSpec((1, 1), lambda: (0, 0)),
        ],
        out_specs=pl.BlockSpec((NGRAPH, H), lambda: (0, 0)),
        out_shape=jax.ShapeDtypeStruct((NGRAPH, H), jnp.float32),
    )(state, w1, b1[None, :], w2.T, b2[None, :])
    return out[:, 0]


_GK = 128


def _sc_gather2(table_a, idx_a, table_b, idx_b):
    e = idx_a.shape[0]
    assert e % _GK == 0 and e == idx_b.shape[0]
    steps = e // _GK
    nw = 32
    per_w = -(-steps // nw)
    mesh = plsc.VectorSubcoreMesh(core_axis_name="c", subcore_axis_name="s")

    @functools.partial(
        pl.kernel, mesh=mesh,
        out_type=(jax.ShapeDtypeStruct((e, H), jnp.float32),
                  jax.ShapeDtypeStruct((e, H), jnp.float32)),
        scratch_types=[
            pltpu.VMEM((_GK,), jnp.int32), pltpu.VMEM((_GK, H), jnp.float32),
            pltpu.VMEM((_GK,), jnp.int32), pltpu.VMEM((_GK, H), jnp.float32),
            pltpu.SemaphoreType.DMA, pltpu.SemaphoreType.DMA,
        ],
    )
    def gk(ta, ia, tb, ib, oa, ob, iva, rva, ivb, rvb, sema, semb):
        wid = lax.axis_index("s") * 2 + lax.axis_index("c")

        def body(j, carry):
            s = wid + j * nw

            @pl.when(s < steps)
            def _():
                base = s * _GK
                pltpu.sync_copy(ia.at[pl.ds(base, _GK)], iva)
                pltpu.sync_copy(ib.at[pl.ds(base, _GK)], ivb)
                ca = pltpu.async_copy(ta.at[iva], rva, sema)
                cb = pltpu.async_copy(tb.at[ivb], rvb, semb)
                ca.wait()
                pltpu.sync_copy(rva, oa.at[pl.ds(base, _GK)])
                cb.wait()
                pltpu.sync_copy(rvb, ob.at[pl.ds(base, _GK)])

            return carry

        lax.fori_loop(0, per_w, body, 0)

    return gk(table_a, idx_a, table_b, idx_b)


def _segsum(m, dst, n):
    return jax.ops.segment_sum(m, dst, num_segments=n)


def kernel(features_rank_0, features_rank_1, features_rank_2,
           adjacencies_rank_0, adjacencies_rank_1,
           incidences_rank_0, incidences_rank_1,
           inv_rr_rank_0, inv_rr_rank_1,
           inv_rrm1_rank_0, inv_rrm1_rank_1,
           batch_rank_0, batch_rank_1, batch_rank_2, params):
    n0 = features_rank_0.shape[0]
    n1 = features_rank_1.shape[0]
    n2 = features_rank_2.shape[0]
    sizes = {'rank_0': n0, 'rank_1': n1, 'rank_2': n2}

    we, be = params['embed']
    h = {
        'rank_0': _embed(features_rank_0, we, be),
        'rank_1': _embed(features_rank_1, we, be),
        'rank_2': _embed(features_rank_2, we, be),
    }

    eb = 1024
    adj = {'rank_0': adjacencies_rank_0, 'rank_1': adjacencies_rank_1}
    inc = {'rank_0': incidences_rank_0, 'rank_1': incidences_rank_1}
    inv_rr = {'rank_0': inv_rr_rank_0, 'rank_1': inv_rr_rank_1}
    inv_rm = {'rank_0': inv_rrm1_rank_0, 'rank_1': inv_rrm1_rank_1}
    upper = {'rank_0': 'rank_1', 'rank_1': 'rank_2'}

    def prep(edges, inv, n_dst):
        src = _pad_rows(edges[0], eb, fill=0)
        dstg = _pad_rows(edges[1], eb, fill=0)
        dsts = _pad_rows(edges[1], eb, fill=n_dst)
        inv8 = _pad_rows(_pad_rows(inv, eb, fill=0).T, 8).T
        return src, dstg, dsts, inv8

    prepped = {}
    for r in ('rank_0', 'rank_1'):
        prepped[(r, 'adj')] = prep(adj[r], inv_rr[r], sizes[r])
        prepped[(r, 'inc')] = prep(inc[r], inv_rm[r], sizes[r])

    for lp in params['layers']:
        h_new = dict(h)
        for r in ('rank_0', 'rank_1'):
            p = lp[r]
            src, dstg, dsts, inv8 = prepped[(r, 'adj')]
            gs, gd = _sc_gather2(h[r], src, h[r], dstg)
            m0 = _edge_mlp(gs, gd, inv8, p['msg_adj'], p['inf_adj'], block=eb)
            agg = _segsum(m0, dsts, sizes[r])
            src2, dstg2, dsts2, inv8b = prepped[(r, 'inc')]
            gs2, gd2 = _sc_gather2(h[upper[r]], src2, h[r], dstg2)
            m1 = _edge_mlp(gs2, gd2, inv8b, p['msg_inc'], p['inf_inc'], block=eb)
            agg = agg + _segsum(m1, dsts2, sizes[r])
            h_new[r] = _update_single(h[r], agg, p['upd'])
        h = h_new

    batches = {'rank_0': batch_rank_0, 'rank_1': batch_rank_1, 'rank_2': batch_rank_2}
    pooled = [_prepool(h[r], batches[r], params['pre_pool'][r])
              for r in ('rank_0', 'rank_1', 'rank_2')]
    state = jnp.concatenate(pooled, axis=1)
    return _postpool(state, params['post_pool'])


# speedup vs baseline: 1.7939x; 1.0016x over previous
import functools

import jax
import jax.numpy as jnp
from jax import lax
from jax.experimental import pallas as pl
from jax.experimental.pallas import tpu as pltpu
from jax.experimental.pallas import tpu_sc as plsc

H = 128
NGRAPH = 32


def _silu(x):
    return x * jax.nn.sigmoid(x)


def _pad_rows(a, m, fill=0):
    n = a.shape[0]
    r = (-n) % m
    if r == 0:
        return a
    pad = [(0, r)] + [(0, 0)] * (a.ndim - 1)
    return jnp.pad(a, pad, constant_values=fill)


def _edge_mlp_body(gs, gd, inv, w1a, w1b, w1c, b1, w2, b2, wiv, bi, out):
    x = (jnp.dot(gs[...], w1a[...], preferred_element_type=jnp.float32)
         + jnp.dot(gd[...], w1b[...], preferred_element_type=jnp.float32)
         + jnp.dot(inv[...], w1c[...], preferred_element_type=jnp.float32)
         + b1[...])
    m = _silu(x)
    m = _silu(jnp.dot(m, w2[...], preferred_element_type=jnp.float32) + b2[...])
    g = jax.nn.sigmoid(jnp.sum(m * wiv[...], axis=1, keepdims=True) + bi[0, 0])
    out[...] = m * g


def _edge_mlp(gs, gd, inv8, mlp, inf, block=1024):
    e = gs.shape[0]
    w1, b1, w2, b2 = mlp
    wi, bi = inf
    grid = (e // block,)
    full = lambda shape: pl.BlockSpec(shape, lambda i: (0, 0))
    return pl.pallas_call(
        _edge_mlp_body,
        grid=grid,
        in_specs=[
            pl.BlockSpec((block, H), lambda i: (i, 0)),
            pl.BlockSpec((block, H), lambda i: (i, 0)),
            pl.BlockSpec((block, 8), lambda i: (i, 0)),
            full((H, H)), full((H, H)), full((8, H)), full((1, H)),
            full((H, H)), full((1, H)),
            full((1, H)), full((1, 1)),
        ],
        out_specs=pl.BlockSpec((block, H), lambda i: (i, 0)),
        out_shape=jax.ShapeDtypeStruct((e, H), jnp.float32),
    )(gs, gd, inv8,
      w1[:H], w1[H:2 * H], _pad_rows(w1[2 * H:], 8), b1[None, :],
      w2, b2[None, :], wi.T, bi[None, :])


def _embed_body(x, we, be, out):
    out[...] = jnp.dot(x[...], we[...], preferred_element_type=jnp.float32) + be[...]


def _embed(x, we, be, block=2048):
    n = x.shape[0]
    xp = _pad_rows(x, block)
    grid = (xp.shape[0] // block,)
    out = pl.pallas_call(
        _embed_body,
        grid=grid,
        in_specs=[
            pl.BlockSpec((block, H), lambda i: (i, 0)),
            pl.BlockSpec((H, H), lambda i: (0, 0)),
            pl.BlockSpec((1, H), lambda i: (0, 0)),
        ],
        out_specs=pl.BlockSpec((block, H), lambda i: (i, 0)),
        out_shape=jax.ShapeDtypeStruct((xp.shape[0], H), jnp.float32),
    )(xp, we, be[None, :])
    return out[:n]


def _update_body(h, agg, w1a, w1b, b1, w2, b2, out):
    x = (jnp.dot(h[...], w1a[...], preferred_element_type=jnp.float32)
         + jnp.dot(agg[...], w1b[...], preferred_element_type=jnp.float32)
         + b1[...])
    out[...] = h[...] + jnp.dot(_silu(x), w2[...], preferred_element_type=jnp.float32) + b2[...]


def _update_single(h, agg, upd, block=2048):
    n = h.shape[0]
    w1, b1, w2, b2 = upd
    hp = _pad_rows(h, block)
    aggp = _pad_rows(agg, block)
    grid = (hp.shape[0] // block,)
    full = lambda shape: pl.BlockSpec(shape, lambda i: (0, 0))
    out = pl.pallas_call(
        _update_body,
        grid=grid,
        in_specs=[
            pl.BlockSpec((block, H), lambda i: (i, 0)),
            pl.BlockSpec((block, H), lambda i: (i, 0)),
            full((H, H)), full((H, H)), full((1, H)), full((H, H)), full((1, H)),
        ],
        out_specs=pl.BlockSpec((block, H), lambda i: (i, 0)),
        out_shape=jax.ShapeDtypeStruct((hp.shape[0], H), jnp.float32),
    )(hp, aggp, w1[:H], w1[H:], b1[None, :], w2, b2[None, :])
    return out[:n]


def _prepool_body(h, bid, w1, b1, w2, b2, out):
    i = pl.program_id(0)

    @pl.when(i == 0)
    def _():
        out[...] = jnp.zeros_like(out)

    x = _silu(jnp.dot(h[...], w1[...], preferred_element_type=jnp.float32) + b1[...])
    x = jnp.dot(x, w2[...], preferred_element_type=jnp.float32) + b2[...]
    ids = bid[...]
    lanes = jnp.arange(NGRAPH, dtype=jnp.int32)[None, :].astype(jnp.float32)
    onehot = (ids == lanes).astype(jnp.float32)
    out[...] += jax.lax.dot_general(onehot, x, (((0,), (0,)), ((), ())),
                                    preferred_element_type=jnp.float32)


def _prepool(h, bid, pre, block=2048):
    w1, b1, w2, b2 = pre
    hp = _pad_rows(h, block)
    bidp = _pad_rows(bid.astype(jnp.float32)[:, None], block, fill=NGRAPH + 1)
    grid = (hp.shape[0] // block,)
    full = lambda shape: pl.BlockSpec(shape, lambda i: (0, 0))
    return pl.pallas_call(
        _prepool_body,
        grid=grid,
        in_specs=[
            pl.BlockSpec((block, H), lambda i: (i, 0)),
            pl.BlockSpec((block, 1), lambda i: (i, 0)),
            full((H, H)), full((1, H)), full((H, H)), full((1, H)),
        ],
        out_specs=pl.BlockSpec((NGRAPH, H), lambda i: (0, 0)),
        out_shape=jax.ShapeDtypeStruct((NGRAPH, H), jnp.float32),
    )(hp, bidp, w1, b1[None, :], w2, b2[None, :])


def _postpool_body(st, w1, b1, w2v, b2, out):
    x = _silu(jnp.dot(st[...], w1[...], preferred_element_type=jnp.float32) + b1[...])
    r = jnp.sum(x * w2v[...], axis=1, keepdims=True) + b2[0, 0]
    out[...] = jnp.broadcast_to(r, (NGRAPH, H))


def _postpool(state, post):
    w1, b1, w2, b2 = post
    out = pl.pallas_call(
        _postpool_body,
        in_specs=[
            pl.BlockSpec((NGRAPH, 3 * H), lambda: (0, 0)),
            pl.BlockSpec((3 * H, H), lambda: (0, 0)),
            pl.BlockSpec((1, H), lambda: (0, 0)),
            pl.BlockSpec((1, H), lambda: (0, 0)),
            pl.BlockSpec((1, 1), lambda: (0, 0)),
        ],
        out_specs=pl.BlockSpec((NGRAPH, H), lambda: (0, 0)),
        out_shape=jax.ShapeDtypeStruct((NGRAPH, H), jnp.float32),
    )(state, w1, b1[None, :], w2.T, b2[None, :])
    return out[:, 0]


_GK = 128


def _sc_gather2(table_a, idx_a, table_b, idx_b):
    e = idx_a.shape[0]
    assert e % _GK == 0 and e == idx_b.shape[0]
    steps = e // _GK
    nw = 32
    per_w = -(-steps // nw)
    mesh = plsc.VectorSubcoreMesh(core_axis_name="c", subcore_axis_name="s")

    @functools.partial(
        pl.kernel, mesh=mesh,
        out_type=(jax.ShapeDtypeStruct((e, H), jnp.float32),
                  jax.ShapeDtypeStruct((e, H), jnp.float32)),
        scratch_types=[
            pltpu.VMEM((_GK,), jnp.int32), pltpu.VMEM((_GK, H), jnp.float32),
            pltpu.VMEM((_GK,), jnp.int32), pltpu.VMEM((_GK, H), jnp.float32),
            pltpu.SemaphoreType.DMA, pltpu.SemaphoreType.DMA,
            pltpu.SemaphoreType.DMA, pltpu.SemaphoreType.DMA,
        ],
    )
    def gk(ta, ia, tb, ib, oa, ob, iva, rva, ivb, rvb, sga, sgb, swa, swb):
        wid = lax.axis_index("s") * 2 + lax.axis_index("c")

        def body(j, carry):
            s = wid + j * nw

            @pl.when(s < steps)
            def _():
                base = s * _GK

                @pl.when(j > 0)
                def _():
                    pltpu.make_async_copy(rva, oa.at[pl.ds(base, _GK)], swa).wait()
                    pltpu.make_async_copy(rvb, ob.at[pl.ds(base, _GK)], swb).wait()

                pltpu.sync_copy(ia.at[pl.ds(base, _GK)], iva)
                pltpu.sync_copy(ib.at[pl.ds(base, _GK)], ivb)
                ca = pltpu.async_copy(ta.at[iva], rva, sga)
                cb = pltpu.async_copy(tb.at[ivb], rvb, sgb)
                ca.wait()
                pltpu.async_copy(rva, oa.at[pl.ds(base, _GK)], swa)
                cb.wait()
                pltpu.async_copy(rvb, ob.at[pl.ds(base, _GK)], swb)

            return carry

        lax.fori_loop(0, per_w, body, 0)

        @pl.when(wid < steps)
        def _():
            pltpu.make_async_copy(rva, oa.at[pl.ds(0, _GK)], swa).wait()
            pltpu.make_async_copy(rvb, ob.at[pl.ds(0, _GK)], swb).wait()

    return gk(table_a, idx_a, table_b, idx_b)


def _segsum(m, dst, n):
    return jax.ops.segment_sum(m, dst, num_segments=n)


def kernel(features_rank_0, features_rank_1, features_rank_2,
           adjacencies_rank_0, adjacencies_rank_1,
           incidences_rank_0, incidences_rank_1,
           inv_rr_rank_0, inv_rr_rank_1,
           inv_rrm1_rank_0, inv_rrm1_rank_1,
           batch_rank_0, batch_rank_1, batch_rank_2, params):
    n0 = features_rank_0.shape[0]
    n1 = features_rank_1.shape[0]
    n2 = features_rank_2.shape[0]
    sizes = {'rank_0': n0, 'rank_1': n1, 'rank_2': n2}

    we, be = params['embed']
    h = {
        'rank_0': _embed(features_rank_0, we, be),
        'rank_1': _embed(features_rank_1, we, be),
        'rank_2': _embed(features_rank_2, we, be),
    }

    eb = 1024
    adj = {'rank_0': adjacencies_rank_0, 'rank_1': adjacencies_rank_1}
    inc = {'rank_0': incidences_rank_0, 'rank_1': incidences_rank_1}
    inv_rr = {'rank_0': inv_rr_rank_0, 'rank_1': inv_rr_rank_1}
    inv_rm = {'rank_0': inv_rrm1_rank_0, 'rank_1': inv_rrm1_rank_1}
    upper = {'rank_0': 'rank_1', 'rank_1': 'rank_2'}

    def prep(edges, inv, n_dst):
        src = _pad_rows(edges[0], eb, fill=0)
        dstg = _pad_rows(edges[1], eb, fill=0)
        dsts = _pad_rows(edges[1], eb, fill=n_dst)
        inv8 = _pad_rows(_pad_rows(inv, eb, fill=0).T, 8).T
        return src, dstg, dsts, inv8

    prepped = {}
    for r in ('rank_0', 'rank_1'):
        prepped[(r, 'adj')] = prep(adj[r], inv_rr[r], sizes[r])
        prepped[(r, 'inc')] = prep(inc[r], inv_rm[r], sizes[r])

    for lp in params['layers']:
        h_new = dict(h)
        for r in ('rank_0', 'rank_1'):
            p = lp[r]
            src, dstg, dsts, inv8 = prepped[(r, 'adj')]
            gs, gd = _sc_gather2(h[r], src, h[r], dstg)
            m0 = _edge_mlp(gs, gd, inv8, p['msg_adj'], p['inf_adj'], block=eb)
            agg = _segsum(m0, dsts, sizes[r])
            src2, dstg2, dsts2, inv8b = prepped[(r, 'inc')]
            gs2, gd2 = _sc_gather2(h[upper[r]], src2, h[r], dstg2)
            m1 = _edge_mlp(gs2, gd2, inv8b, p['msg_inc'], p['inf_inc'], block=eb)
            agg = agg + _segsum(m1, dsts2, sizes[r])
            h_new[r] = _update_single(h[r], agg, p['upd'])
        h = h_new

    batches = {'rank_0': batch_rank_0, 'rank_1': batch_rank_1, 'rank_2': batch_rank_2}
    pooled = [_prepool(h[r], batches[r], params['pre_pool'][r])
              for r in ('rank_0', 'rank_1', 'rank_2')]
    state = jnp.concatenate(pooled, axis=1)
    return _postpool(state, params['post_pool'])


# speedup vs baseline: 1.8278x; 1.0189x over previous
import functools

import jax
import jax.numpy as jnp
from jax import lax
from jax.experimental import pallas as pl
from jax.experimental.pallas import tpu as pltpu
from jax.experimental.pallas import tpu_sc as plsc

H = 128
NGRAPH = 32


def _silu(x):
    return x * jax.nn.sigmoid(x)


def _pad_rows(a, m, fill=0):
    n = a.shape[0]
    r = (-n) % m
    if r == 0:
        return a
    pad = [(0, r)] + [(0, 0)] * (a.ndim - 1)
    return jnp.pad(a, pad, constant_values=fill)


def _edge_mlp_body(gs, gd, inv, w1a, w1b, w1c, b1, w2, b2, wiv, bi, out):
    x = (jnp.dot(gs[...], w1a[...], preferred_element_type=jnp.float32)
         + jnp.dot(gd[...], w1b[...], preferred_element_type=jnp.float32)
         + jnp.dot(inv[...], w1c[...], preferred_element_type=jnp.float32)
         + b1[...])
    m = _silu(x)
    m = _silu(jnp.dot(m, w2[...], preferred_element_type=jnp.float32) + b2[...])
    g = jax.nn.sigmoid(jnp.sum(m * wiv[...], axis=1, keepdims=True) + bi[0, 0])
    out[...] = m * g


def _edge_mlp(gs, gd, inv8, mlp, inf, block=1024):
    e = gs.shape[0]
    w1, b1, w2, b2 = mlp
    wi, bi = inf
    grid = (e // block,)
    full = lambda shape: pl.BlockSpec(shape, lambda i: (0, 0))
    return pl.pallas_call(
        _edge_mlp_body,
        grid=grid,
        in_specs=[
            pl.BlockSpec((block, H), lambda i: (i, 0)),
            pl.BlockSpec((block, H), lambda i: (i, 0)),
            pl.BlockSpec((block, 8), lambda i: (i, 0)),
            full((H, H)), full((H, H)), full((8, H)), full((1, H)),
            full((H, H)), full((1, H)),
            full((1, H)), full((1, 1)),
        ],
        out_specs=pl.BlockSpec((block, H), lambda i: (i, 0)),
        out_shape=jax.ShapeDtypeStruct((e, H), jnp.float32),
    )(gs, gd, inv8,
      w1[:H], w1[H:2 * H], _pad_rows(w1[2 * H:], 8), b1[None, :],
      w2, b2[None, :], wi.T, bi[None, :])


def _embed_body(x, we, be, out):
    out[...] = jnp.dot(x[...], we[...], preferred_element_type=jnp.float32) + be[...]


def _embed(x, we, be, block=2048):
    n = x.shape[0]
    xp = _pad_rows(x, block)
    grid = (xp.shape[0] // block,)
    out = pl.pallas_call(
        _embed_body,
        grid=grid,
        in_specs=[
            pl.BlockSpec((block, H), lambda i: (i, 0)),
            pl.BlockSpec((H, H), lambda i: (0, 0)),
            pl.BlockSpec((1, H), lambda i: (0, 0)),
        ],
        out_specs=pl.BlockSpec((block, H), lambda i: (i, 0)),
        out_shape=jax.ShapeDtypeStruct((xp.shape[0], H), jnp.float32),
    )(xp, we, be[None, :])
    return out[:n]


def _update_body(h, agg, w1a, w1b, b1, w2, b2, out):
    x = (jnp.dot(h[...], w1a[...], preferred_element_type=jnp.float32)
         + jnp.dot(agg[...], w1b[...], preferred_element_type=jnp.float32)
         + b1[...])
    out[...] = h[...] + jnp.dot(_silu(x), w2[...], preferred_element_type=jnp.float32) + b2[...]


def _update_single(h, agg, upd, block=2048):
    n = h.shape[0]
    w1, b1, w2, b2 = upd
    hp = _pad_rows(h, block)
    aggp = _pad_rows(agg, block)
    grid = (hp.shape[0] // block,)
    full = lambda shape: pl.BlockSpec(shape, lambda i: (0, 0))
    out = pl.pallas_call(
        _update_body,
        grid=grid,
        in_specs=[
            pl.BlockSpec((block, H), lambda i: (i, 0)),
            pl.BlockSpec((block, H), lambda i: (i, 0)),
            full((H, H)), full((H, H)), full((1, H)), full((H, H)), full((1, H)),
        ],
        out_specs=pl.BlockSpec((block, H), lambda i: (i, 0)),
        out_shape=jax.ShapeDtypeStruct((hp.shape[0], H), jnp.float32),
    )(hp, aggp, w1[:H], w1[H:], b1[None, :], w2, b2[None, :])
    return out[:n]


def _prepool_body(h, bid, w1, b1, w2, b2, out):
    i = pl.program_id(0)

    @pl.when(i == 0)
    def _():
        out[...] = jnp.zeros_like(out)

    x = _silu(jnp.dot(h[...], w1[...], preferred_element_type=jnp.float32) + b1[...])
    x = jnp.dot(x, w2[...], preferred_element_type=jnp.float32) + b2[...]
    ids = bid[...]
    lanes = jnp.arange(NGRAPH, dtype=jnp.int32)[None, :].astype(jnp.float32)
    onehot = (ids == lanes).astype(jnp.float32)
    out[...] += jax.lax.dot_general(onehot, x, (((0,), (0,)), ((), ())),
                                    preferred_element_type=jnp.float32)


def _prepool(h, bid, pre, block=2048):
    w1, b1, w2, b2 = pre
    hp = _pad_rows(h, block)
    bidp = _pad_rows(bid.astype(jnp.float32)[:, None], block, fill=NGRAPH + 1)
    grid = (hp.shape[0] // block,)
    full = lambda shape: pl.BlockSpec(shape, lambda i: (0, 0))
    return pl.pallas_call(
        _prepool_body,
        grid=grid,
        in_specs=[
            pl.BlockSpec((block, H), lambda i: (i, 0)),
            pl.BlockSpec((block, 1), lambda i: (i, 0)),
            full((H, H)), full((1, H)), full((H, H)), full((1, H)),
        ],
        out_specs=pl.BlockSpec((NGRAPH, H), lambda i: (0, 0)),
        out_shape=jax.ShapeDtypeStruct((NGRAPH, H), jnp.float32),
    )(hp, bidp, w1, b1[None, :], w2, b2[None, :])


def _postpool_body(st, w1, b1, w2v, b2, out):
    x = _silu(jnp.dot(st[...], w1[...], preferred_element_type=jnp.float32) + b1[...])
    r = jnp.sum(x * w2v[...], axis=1, keepdims=True) + b2[0, 0]
    out[...] = jnp.broadcast_to(r, (NGRAPH, H))


def _postpool(state, post):
    w1, b1, w2, b2 = post
    out = pl.pallas_call(
        _postpool_body,
        in_specs=[
            pl.BlockSpec((NGRAPH, 3 * H), lambda: (0, 0)),
            pl.BlockSpec((3 * H, H), lambda: (0, 0)),
            pl.BlockSpec((1, H), lambda: (0, 0)),
            pl.BlockSpec((1, H), lambda: (0, 0)),
            pl.BlockSpec((1, 1), lambda: (0, 0)),
        ],
        out_specs=pl.BlockSpec((NGRAPH, H), lambda: (0, 0)),
        out_shape=jax.ShapeDtypeStruct((NGRAPH, H), jnp.float32),
    )(state, w1, b1[None, :], w2.T, b2[None, :])
    return out[:, 0]


_GK = 128


def _sc_gather2(table_a, idx_a, table_b, idx_b):
    e = idx_a.shape[0]
    assert e % _GK == 0 and e == idx_b.shape[0]
    steps = e // _GK
    nv = 64
    per_w = -(-steps // nv)
    mesh = plsc.VectorSubcoreMesh(core_axis_name="c", subcore_axis_name="s")

    @functools.partial(
        pl.kernel, mesh=mesh,
        out_type=(jax.ShapeDtypeStruct((e, H), jnp.float32),
                  jax.ShapeDtypeStruct((e, H), jnp.float32)),
        scratch_types=[
            pltpu.VMEM((_GK,), jnp.int32), pltpu.VMEM((_GK, H), jnp.float32),
            pltpu.VMEM((_GK,), jnp.int32), pltpu.VMEM((_GK, H), jnp.float32),
            pltpu.VMEM((_GK,), jnp.int32), pltpu.VMEM((_GK, H), jnp.float32),
            pltpu.VMEM((_GK,), jnp.int32), pltpu.VMEM((_GK, H), jnp.float32),
            pltpu.SemaphoreType.DMA, pltpu.SemaphoreType.DMA,
            pltpu.SemaphoreType.DMA, pltpu.SemaphoreType.DMA,
            pltpu.SemaphoreType.DMA, pltpu.SemaphoreType.DMA,
            pltpu.SemaphoreType.DMA, pltpu.SemaphoreType.DMA,
        ],
    )
    def gk(ta, ia, tb, ib, oa, ob,
           iva0, rva0, ivb0, rvb0, iva1, rva1, ivb1, rvb1,
           sga0, sgb0, sga1, sgb1, swa0, swb0, swa1, swb1):
        wid = lax.axis_index("s") * 2 + lax.axis_index("c")

        def body(j, carry):
            s0 = 2 * wid + j * nv
            s1 = s0 + 1

            @pl.when(s0 < steps)
            def _():
                base = s0 * _GK

                @pl.when(j > 0)
                def _():
                    pltpu.make_async_copy(rva0, oa.at[pl.ds(base, _GK)], swa0).wait()
                    pltpu.make_async_copy(rvb0, ob.at[pl.ds(base, _GK)], swb0).wait()

                pltpu.sync_copy(ia.at[pl.ds(base, _GK)], iva0)
                pltpu.sync_copy(ib.at[pl.ds(base, _GK)], ivb0)
                pltpu.async_copy(ta.at[iva0], rva0, sga0)
                pltpu.async_copy(tb.at[ivb0], rvb0, sgb0)

            @pl.when(s1 < steps)
            def _():
                base = s1 * _GK

                @pl.when(j > 0)
                def _():
                    pltpu.make_async_copy(rva1, oa.at[pl.ds(base, _GK)], swa1).wait()
                    pltpu.make_async_copy(rvb1, ob.at[pl.ds(base, _GK)], swb1).wait()

                pltpu.sync_copy(ia.at[pl.ds(base, _GK)], iva1)
                pltpu.sync_copy(ib.at[pl.ds(base, _GK)], ivb1)
                pltpu.async_copy(ta.at[iva1], rva1, sga1)
                pltpu.async_copy(tb.at[ivb1], rvb1, sgb1)

            @pl.when(s0 < steps)
            def _():
                base = s0 * _GK
                pltpu.make_async_copy(ta.at[iva0], rva0, sga0).wait()
                pltpu.async_copy(rva0, oa.at[pl.ds(base, _GK)], swa0)
                pltpu.make_async_copy(tb.at[ivb0], rvb0, sgb0).wait()
                pltpu.async_copy(rvb0, ob.at[pl.ds(base, _GK)], swb0)

            @pl.when(s1 < steps)
            def _():
                base = s1 * _GK
                pltpu.make_async_copy(ta.at[iva1], rva1, sga1).wait()
                pltpu.async_copy(rva1, oa.at[pl.ds(base, _GK)], swa1)
                pltpu.make_async_copy(tb.at[ivb1], rvb1, sgb1).wait()
                pltpu.async_copy(rvb1, ob.at[pl.ds(base, _GK)], swb1)

            return carry

        lax.fori_loop(0, per_w, body, 0)

        @pl.when(2 * wid < steps)
        def _():
            pltpu.make_async_copy(rva0, oa.at[pl.ds(0, _GK)], swa0).wait()
            pltpu.make_async_copy(rvb0, ob.at[pl.ds(0, _GK)], swb0).wait()

        @pl.when(2 * wid + 1 < steps)
        def _():
            pltpu.make_async_copy(rva1, oa.at[pl.ds(0, _GK)], swa1).wait()
            pltpu.make_async_copy(rvb1, ob.at[pl.ds(0, _GK)], swb1).wait()

    return gk(table_a, idx_a, table_b, idx_b)


def _segsum(m, dst, n):
    return jax.ops.segment_sum(m, dst, num_segments=n)


def kernel(features_rank_0, features_rank_1, features_rank_2,
           adjacencies_rank_0, adjacencies_rank_1,
           incidences_rank_0, incidences_rank_1,
           inv_rr_rank_0, inv_rr_rank_1,
           inv_rrm1_rank_0, inv_rrm1_rank_1,
           batch_rank_0, batch_rank_1, batch_rank_2, params):
    n0 = features_rank_0.shape[0]
    n1 = features_rank_1.shape[0]
    n2 = features_rank_2.shape[0]
    sizes = {'rank_0': n0, 'rank_1': n1, 'rank_2': n2}

    we, be = params['embed']
    h = {
        'rank_0': _embed(features_rank_0, we, be),
        'rank_1': _embed(features_rank_1, we, be),
        'rank_2': _embed(features_rank_2, we, be),
    }

    eb = 1024
    adj = {'rank_0': adjacencies_rank_0, 'rank_1': adjacencies_rank_1}
    inc = {'rank_0': incidences_rank_0, 'rank_1': incidences_rank_1}
    inv_rr = {'rank_0': inv_rr_rank_0, 'rank_1': inv_rr_rank_1}
    inv_rm = {'rank_0': inv_rrm1_rank_0, 'rank_1': inv_rrm1_rank_1}
    upper = {'rank_0': 'rank_1', 'rank_1': 'rank_2'}

    def prep(edges, inv, n_dst):
        src = _pad_rows(edges[0], eb, fill=0)
        dstg = _pad_rows(edges[1], eb, fill=0)
        dsts = _pad_rows(edges[1], eb, fill=n_dst)
        inv8 = _pad_rows(_pad_rows(inv, eb, fill=0).T, 8).T
        return src, dstg, dsts, inv8

    prepped = {}
    for r in ('rank_0', 'rank_1'):
        prepped[(r, 'adj')] = prep(adj[r], inv_rr[r], sizes[r])
        prepped[(r, 'inc')] = prep(inc[r], inv_rm[r], sizes[r])

    for lp in params['layers']:
        h_new = dict(h)
        for r in ('rank_0', 'rank_1'):
            p = lp[r]
            src, dstg, dsts, inv8 = prepped[(r, 'adj')]
            gs, gd = _sc_gather2(h[r], src, h[r], dstg)
            m0 = _edge_mlp(gs, gd, inv8, p['msg_adj'], p['inf_adj'], block=eb)
            agg = _segsum(m0, dsts, sizes[r])
            src2, dstg2, dsts2, inv8b = prepped[(r, 'inc')]
            gs2, gd2 = _sc_gather2(h[upper[r]], src2, h[r], dstg2)
            m1 = _edge_mlp(gs2, gd2, inv8b, p['msg_inc'], p['inf_inc'], block=eb)
            agg = agg + _segsum(m1, dsts2, sizes[r])
            h_new[r] = _update_single(h[r], agg, p['upd'])
        h = h_new

    batches = {'rank_0': batch_rank_0, 'rank_1': batch_rank_1, 'rank_2': batch_rank_2}
    pooled = [_prepool(h[r], batches[r], params['pre_pool'][r])
              for r in ('rank_0', 'rank_1', 'rank_2')]
    state = jnp.concatenate(pooled, axis=1)
    return _postpool(state, params['post_pool'])
